# baseline (speedup 1.0000x reference)
import functools

import jax
import jax.numpy as jnp
from jax import lax
from jax.experimental import pallas as pl
from jax.experimental.pallas import tpu as pltpu
from jax.experimental.pallas import tpu_sc as plsc

LANES = 16
GCHUNK = 128
GPB = 2
CHUNK = GCHUNK * GPB
NBUF = 3


def _make_sc_lookup(vocab, d, n_rows):
  info = plsc.get_sparse_core_info()
  nw = info.num_cores * info.num_subcores
  assert n_rows % (nw * CHUNK) == 0
  per_w = n_rows // nw
  ngc = per_w // GCHUNK
  ncs = per_w // CHUNK
  assert ncs >= 5
  n_main = (ncs - 2 - NBUF) // NBUF * NBUF
  tail_lo = 2 + n_main

  mesh = plsc.VectorSubcoreMesh(core_axis_name="c", subcore_axis_name="s")
  out_sd = jax.ShapeDtypeStruct((n_rows, d), jnp.float32)

  @functools.partial(
      pl.kernel,
      out_type=(out_sd, out_sd),
      mesh=mesh,
      scratch_types=[
          pltpu.VMEM((ngc, GCHUNK), jnp.int32),
          pltpu.VMEM((ngc, GCHUNK), jnp.int32),
          pltpu.VMEM((CHUNK, d), jnp.float32),
          pltpu.VMEM((CHUNK, d), jnp.float32),
          pltpu.VMEM((CHUNK, d), jnp.float32),
          pltpu.VMEM((LANES,), jnp.float32),
          pltpu.VMEM((LANES,), jnp.float32),
          pltpu.SemaphoreType.DMA,
          pltpu.SemaphoreType.DMA,
          pltpu.SemaphoreType.DMA,
          pltpu.SemaphoreType.DMA,
          pltpu.SemaphoreType.DMA,
          pltpu.SemaphoreType.DMA,
      ],
  )
  def sc_lookup(enc_idx, dec_idx, enc_scale, dec_scale, table,
                enc_out, dec_out,
                enc_idx_v, dec_idx_v, buf0, buf1, buf2, enc_sc_v, dec_sc_v,
                g0, g1, g2, o0, o1, o2):
    bufs = (buf0, buf1, buf2)
    gsems = (g0, g1, g2)
    osems = (o0, o1, o2)

    wid = lax.axis_index("s") * info.num_cores + lax.axis_index("c")
    base = wid * per_w

    pltpu.sync_copy(enc_idx.at[wid], enc_idx_v)
    pltpu.sync_copy(dec_idx.at[wid], dec_idx_v)
    pltpu.sync_copy(enc_scale, enc_sc_v)
    pltpu.sync_copy(dec_scale, dec_sc_v)

    def run_stream(idx_v, sc_v, out):
      sc = sc_v[...]

      def start_gather(c, b):
        for g in range(GPB):
          pltpu.async_copy(table.at[idx_v.at[c * GPB + g]],
                           bufs[b].at[pl.ds(g * GCHUNK, GCHUNK)], gsems[b])

      def wait_gather(c, b):
        for g in range(GPB):
          pltpu.make_async_copy(table.at[idx_v.at[c * GPB + g]],
                                bufs[b].at[pl.ds(g * GCHUNK, GCHUNK)],
                                gsems[b]).wait()

      def start_scatter(c, b):
        pltpu.async_copy(bufs[b], out.at[pl.ds(base + c * CHUNK, CHUNK)],
                         osems[b])

      def wait_scatter(c, b):
        pltpu.make_async_copy(bufs[b], out.at[pl.ds(base + c * CHUNK, CHUNK)],
                              osems[b]).wait()

      def scale(b):
        buf = bufs[b]

        @pl.loop(0, CHUNK)
        def _row(i):
          for k in range(d // LANES):
            buf[i, pl.ds(k * LANES, LANES)] = (
                buf[i, pl.ds(k * LANES, LANES)] * sc)

      start_gather(0, 0)
      start_gather(1, 1)
      wait_gather(0, 0)
      scale(0)
      start_scatter(0, 0)
      start_gather(2, 2)
      wait_gather(1, 1)
      scale(1)
      start_scatter(1, 1)
      wait_scatter(0, 0)
      start_gather(3, 0)

      @pl.loop(2, tail_lo, step=NBUF)
      def _main(j):
        for t in range(NBUF):
          c = j + t
          b = (2 + t) % NBUF
          wait_gather(c, b)
          scale(b)
          start_scatter(c, b)
          wait_scatter(c - 1, (b + 2) % NBUF)
          start_gather(c + 2, (b + 2) % NBUF)

      for c in range(tail_lo, ncs):
        b = c % NBUF
        wait_gather(c, b)
        scale(b)
        start_scatter(c, b)
        wait_scatter(c - 1, (c - 1) % NBUF)
        if c + 2 < ncs:
          start_gather(c + 2, (c + 2) % NBUF)
      wait_scatter(ncs - 1, (ncs - 1) % NBUF)

    run_stream(enc_idx_v, enc_sc_v, enc_out)
    run_stream(dec_idx_v, dec_sc_v, dec_out)

  return sc_lookup, nw, ngc


def kernel(input_ids, encoder_embed_scale, decoder_input_ids,
           decoder_embed_scale, shared_weight):
  b, l = input_ids.shape
  vocab, d = shared_weight.shape
  n_rows = b * l

  sc_lookup, nw, ngc = _make_sc_lookup(vocab, d, n_rows)

  enc_idx = input_ids.astype(jnp.int32).reshape(nw, ngc, GCHUNK)
  dec_idx = decoder_input_ids.astype(jnp.int32).reshape(nw, ngc, GCHUNK)
  enc_s = jnp.broadcast_to(encoder_embed_scale.astype(jnp.float32), (LANES,))
  dec_s = jnp.broadcast_to(decoder_embed_scale.astype(jnp.float32), (LANES,))

  enc_out, dec_out = sc_lookup(enc_idx, dec_idx, enc_s, dec_s, shared_weight)
  return (enc_out.reshape(b, l, d), dec_out.reshape(b, l, d))


# speedup vs baseline: 1.0098x; 1.0098x over previous
import functools

import jax
import jax.numpy as jnp
from jax import lax
from jax.experimental import pallas as pl
from jax.experimental.pallas import tpu as pltpu
from jax.experimental.pallas import tpu_sc as plsc

LANES = 16
CHUNK = 128
NBUF = 3


def _make_sc_lookup(vocab, d, n_rows):
  info = plsc.get_sparse_core_info()
  nw = info.num_cores * info.num_subcores
  assert n_rows % (nw * CHUNK) == 0
  per_w = n_rows // nw
  ncs = per_w // CHUNK
  assert ncs >= 5
  n_main = (ncs - 2 - NBUF) // NBUF * NBUF
  tail_lo = 2 + n_main

  mesh = plsc.VectorSubcoreMesh(core_axis_name="c", subcore_axis_name="s")
  out_sd = jax.ShapeDtypeStruct((n_rows, d), jnp.float32)

  @functools.partial(
      pl.kernel,
      out_type=(out_sd, out_sd),
      mesh=mesh,
      scratch_types=[
          pltpu.VMEM((per_w,), jnp.int32),
          pltpu.VMEM((per_w,), jnp.int32),
          pltpu.VMEM((2, LANES), jnp.float32),
          pltpu.VMEM((CHUNK, d), jnp.float32),
          pltpu.VMEM((CHUNK, d), jnp.float32),
          pltpu.VMEM((CHUNK, d), jnp.float32),
          pltpu.SemaphoreType.DMA,
          pltpu.SemaphoreType.DMA,
          pltpu.SemaphoreType.DMA,
          pltpu.SemaphoreType.DMA,
          pltpu.SemaphoreType.DMA,
          pltpu.SemaphoreType.DMA,
      ],
  )
  def sc_lookup(enc_idx, dec_idx, scales, table,
                enc_out, dec_out,
                enc_idx_v, dec_idx_v, sc_v, buf0, buf1, buf2,
                g0, g1, g2, o0, o1, o2):
    bufs = (buf0, buf1, buf2)
    gsems = (g0, g1, g2)
    osems = (o0, o1, o2)

    wid = lax.axis_index("s") * info.num_cores + lax.axis_index("c")
    base = wid * per_w

    pltpu.sync_copy(enc_idx.at[pl.ds(wid * per_w, per_w)], enc_idx_v)
    pltpu.sync_copy(dec_idx.at[pl.ds(wid * per_w, per_w)], dec_idx_v)
    pltpu.sync_copy(scales, sc_v)

    def run_stream(idx_v, sc, out):

      def start_gather(c, b):
        pltpu.async_copy(table.at[idx_v.at[pl.ds(c * CHUNK, CHUNK)]],
                         bufs[b], gsems[b])

      def wait_gather(c, b):
        pltpu.make_async_copy(table.at[idx_v.at[pl.ds(c * CHUNK, CHUNK)]],
                              bufs[b], gsems[b]).wait()

      def start_scatter(c, b):
        pltpu.async_copy(bufs[b], out.at[pl.ds(base + c * CHUNK, CHUNK)],
                         osems[b])

      def wait_scatter(c, b):
        pltpu.make_async_copy(bufs[b], out.at[pl.ds(base + c * CHUNK, CHUNK)],
                              osems[b]).wait()

      def scale(b):
        buf = bufs[b]

        @pl.loop(0, CHUNK)
        def _row(i):
          for k in range(d // LANES):
            buf[i, pl.ds(k * LANES, LANES)] = (
                buf[i, pl.ds(k * LANES, LANES)] * sc)

      start_gather(0, 0)
      start_gather(1, 1)
      wait_gather(0, 0)
      scale(0)
      start_scatter(0, 0)
      start_gather(2, 2)
      wait_gather(1, 1)
      scale(1)
      start_scatter(1, 1)
      wait_scatter(0, 0)
      start_gather(3, 0)

      @pl.loop(2, tail_lo, step=NBUF)
      def _main(j):
        for t in range(NBUF):
          c = j + t
          b = (2 + t) % NBUF
          wait_gather(c, b)
          scale(b)
          start_scatter(c, b)
          wait_scatter(c - 1, (b + 2) % NBUF)
          start_gather(c + 2, (b + 2) % NBUF)

      for c in range(tail_lo, ncs):
        b = c % NBUF
        wait_gather(c, b)
        scale(b)
        start_scatter(c, b)
        wait_scatter(c - 1, (c - 1) % NBUF)
        if c + 2 < ncs:
          start_gather(c + 2, (c + 2) % NBUF)
      wait_scatter(ncs - 1, (ncs - 1) % NBUF)

    run_stream(enc_idx_v, sc_v[0], enc_out)
    run_stream(dec_idx_v, sc_v[1], dec_out)

  return sc_lookup, nw, ncs


def kernel(input_ids, encoder_embed_scale, decoder_input_ids,
           decoder_embed_scale, shared_weight):
  b, l = input_ids.shape
  vocab, d = shared_weight.shape
  n_rows = b * l

  sc_lookup, nw, ncs = _make_sc_lookup(vocab, d, n_rows)

  enc_idx = input_ids.astype(jnp.int32).reshape(-1)
  dec_idx = decoder_input_ids.astype(jnp.int32).reshape(-1)
  scales = jnp.broadcast_to(
      jnp.stack([encoder_embed_scale, decoder_embed_scale]).astype(
          jnp.float32)[:, None], (2, LANES))

  enc_out, dec_out = sc_lookup(enc_idx, dec_idx, scales, shared_weight)
  return (enc_out.reshape(b, l, d), dec_out.reshape(b, l, d))


# speedup vs baseline: 1.0149x; 1.0050x over previous
import functools

import jax
import jax.numpy as jnp
from jax import lax
from jax.experimental import pallas as pl
from jax.experimental.pallas import tpu as pltpu
from jax.experimental.pallas import tpu_sc as plsc

LANES = 16
CHUNK = 128
NBUF = 3


def _make_sc_lookup(vocab, d, n_rows):
  info = plsc.get_sparse_core_info()
  nw = info.num_cores * info.num_subcores
  assert n_rows % (nw * CHUNK) == 0
  per_w = n_rows // nw
  ncs = per_w // CHUNK
  assert ncs >= 5
  n_main = (ncs - 2 - NBUF) // NBUF * NBUF
  tail_lo = 2 + n_main

  mesh = plsc.VectorSubcoreMesh(core_axis_name="c", subcore_axis_name="s")
  out_sd = jax.ShapeDtypeStruct((n_rows, d), jnp.float32)

  @functools.partial(
      pl.kernel,
      out_type=(out_sd, out_sd),
      mesh=mesh,
      scratch_types=[
          pltpu.VMEM((per_w,), jnp.int32),
          pltpu.VMEM((per_w,), jnp.int32),
          pltpu.VMEM((2, LANES), jnp.float32),
          pltpu.VMEM((CHUNK, d), jnp.float32),
          pltpu.VMEM((CHUNK, d), jnp.float32),
          pltpu.VMEM((CHUNK, d), jnp.float32),
          pltpu.SemaphoreType.DMA,
          pltpu.SemaphoreType.DMA,
          pltpu.SemaphoreType.DMA,
          pltpu.SemaphoreType.DMA,
          pltpu.SemaphoreType.DMA,
          pltpu.SemaphoreType.DMA,
          pltpu.SemaphoreType.DMA,
      ],
  )
  def sc_lookup(enc_idx, dec_idx, scales, table,
                enc_out, dec_out,
                enc_idx_v, dec_idx_v, sc_v, buf0, buf1, buf2,
                g0, g1, g2, o0, o1, o2, dsem):
    bufs = (buf0, buf1, buf2)
    gsems = (g0, g1, g2)
    osems = (o0, o1, o2)

    wid = lax.axis_index("s") * info.num_cores + lax.axis_index("c")
    base = wid * per_w

    pltpu.sync_copy(enc_idx.at[pl.ds(wid * per_w, per_w)], enc_idx_v)
    dec_stage = pltpu.async_copy(dec_idx.at[pl.ds(wid * per_w, per_w)],
                                 dec_idx_v, dsem)
    pltpu.sync_copy(scales, sc_v)

    def run_stream(idx_v, sc, out):

      def start_gather(c, b):
        pltpu.async_copy(table.at[idx_v.at[pl.ds(c * CHUNK, CHUNK)]],
                         bufs[b], gsems[b])

      def wait_gather(c, b):
        pltpu.make_async_copy(table.at[idx_v.at[pl.ds(c * CHUNK, CHUNK)]],
                              bufs[b], gsems[b]).wait()

      def start_scatter(c, b):
        pltpu.async_copy(bufs[b], out.at[pl.ds(base + c * CHUNK, CHUNK)],
                         osems[b])

      def wait_scatter(c, b):
        pltpu.make_async_copy(bufs[b], out.at[pl.ds(base + c * CHUNK, CHUNK)],
                              osems[b]).wait()

      def scale(b):
        buf = bufs[b]

        @pl.loop(0, CHUNK)
        def _row(i):
          for k in range(d // LANES):
            buf[i, pl.ds(k * LANES, LANES)] = (
                buf[i, pl.ds(k * LANES, LANES)] * sc)

      start_gather(0, 0)
      start_gather(1, 1)
      wait_gather(0, 0)
      scale(0)
      start_scatter(0, 0)
      start_gather(2, 2)
      wait_gather(1, 1)
      scale(1)
      start_scatter(1, 1)
      wait_scatter(0, 0)
      start_gather(3, 0)

      @pl.loop(2, tail_lo, step=NBUF)
      def _main(j):
        for t in range(NBUF):
          c = j + t
          b = (2 + t) % NBUF
          wait_gather(c, b)
          scale(b)
          start_scatter(c, b)
          wait_scatter(c - 1, (b + 2) % NBUF)
          start_gather(c + 2, (b + 2) % NBUF)

      for c in range(tail_lo, ncs):
        b = c % NBUF
        wait_gather(c, b)
        scale(b)
        start_scatter(c, b)
        wait_scatter(c - 1, (c - 1) % NBUF)
        if c + 2 < ncs:
          start_gather(c + 2, (c + 2) % NBUF)
      wait_scatter(ncs - 1, (ncs - 1) % NBUF)

    run_stream(enc_idx_v, sc_v[0], enc_out)
    dec_stage.wait()
    run_stream(dec_idx_v, sc_v[1], dec_out)

  return sc_lookup, nw, ncs


def kernel(input_ids, encoder_embed_scale, decoder_input_ids,
           decoder_embed_scale, shared_weight):
  b, l = input_ids.shape
  vocab, d = shared_weight.shape
  n_rows = b * l

  sc_lookup, nw, ncs = _make_sc_lookup(vocab, d, n_rows)

  enc_idx = input_ids.astype(jnp.int32).reshape(-1)
  dec_idx = decoder_input_ids.astype(jnp.int32).reshape(-1)
  scales = jnp.broadcast_to(
      jnp.stack([encoder_embed_scale, decoder_embed_scale]).astype(
          jnp.float32)[:, None], (2, LANES))

  enc_out, dec_out = sc_lookup(enc_idx, dec_idx, scales, shared_weight)
  return (enc_out.reshape(b, l, d), dec_out.reshape(b, l, d))
